# baseline (speedup 1.0000x reference)
import functools

import jax
import jax.numpy as jnp
from jax import lax
from jax.experimental import pallas as pl
from jax.experimental.pallas import tpu as pltpu
from jax.experimental.pallas import tpu_sc as plsc

_B, _F, _NC, _P = 8, 256, 1024, 576
_SC_NC, _SC_NS, _L = 2, 16, 16
_NW = _SC_NC * _SC_NS
_FPW = _F // _NW
_PC = _P // _L


def _tc_body(x_ref, c_ref, y_ref, loss_ref, csq_ref):
    b = pl.program_id(0)
    nb = pl.num_programs(0)
    xb = x_ref[0]
    C = c_ref[...]

    @pl.when(b == 0)
    def _():
        csq_ref[...] = jnp.sum(C * C, axis=0, keepdims=True)

    c_sq = csq_ref[...]
    S2 = jax.lax.dot_general(C, xb, (((0,), (0,)), ((), ())),
                             preferred_element_type=jnp.float32)
    score = c_sq.T - 2.0 * S2
    m = jnp.min(score, axis=0, keepdims=True)
    iota0 = jax.lax.broadcasted_iota(jnp.int32, (_NC, _P), 0)
    idx = jnp.min(jnp.where(score == m, iota0, _NC), axis=0)
    oh = (iota0 == idx[None, :]).astype(jnp.float32)
    xq = jax.lax.dot_general(C, oh, (((1,), (0,)), ((), ())),
                             preferred_element_type=jnp.float32)
    y_ref[0] = xq
    x_sq = jnp.sum(xb * xb, axis=0)
    partial = jnp.sum(x_sq + m[0])

    @pl.when(b == 0)
    def _():
        loss_ref[0, 0] = 0.0

    loss_ref[0, 0] += partial

    @pl.when(b == nb - 1)
    def _():
        loss_ref[0, 0] = loss_ref[0, 0] / (_B * _F * _P)


def _sc_gather(c_hbm, idx_hbm, y_hbm, cflat, idxv, obuf):
    wid = lax.axis_index("s") * _SC_NC + lax.axis_index("c")
    f0 = wid * _FPW
    pltpu.sync_copy(c_hbm.at[pl.ds(f0 * _NC, _FPW * _NC)], cflat)
    pltpu.sync_copy(idx_hbm, idxv)

    def chunk(t, _):
        b = t // _PC
        c = t % _PC
        iv = idxv[b, pl.ds(c * _L, _L)]
        for fl in range(_FPW):
            vals = plsc.load_gather(cflat, [iv + fl * _NC])
            obuf[b, fl, pl.ds(c * _L, _L)] = vals
        return 0

    lax.fori_loop(0, _B * _PC, chunk, 0)
    pltpu.sync_copy(obuf, y_hbm.at[:, pl.ds(f0, _FPW), :])


def kernel(x, centroids):
    x3 = x.reshape(_B, _F, _P)
    y, loss = pl.pallas_call(
        _tc_body,
        grid=(_B,),
        in_specs=[
            pl.BlockSpec((1, _F, _P), lambda b: (b, 0, 0)),
            pl.BlockSpec((_F, _NC), lambda b: (0, 0)),
        ],
        out_specs=[
            pl.BlockSpec((1, _F, _P), lambda b: (b, 0, 0)),
            pl.BlockSpec(memory_space=pltpu.SMEM, block_shape=(1, 1),
                         index_map=lambda b: (0, 0)),
        ],
        out_shape=[
            jax.ShapeDtypeStruct((_B, _F, _P), jnp.float32),
            jax.ShapeDtypeStruct((1, 1), jnp.float32),
        ],
        scratch_shapes=[pltpu.VMEM((1, _NC), jnp.float32)],
        compiler_params=pltpu.CompilerParams(
            dimension_semantics=("arbitrary",),
            fuse_transposed_lhs_in_matmul=True,
        ),
    )(x3, centroids)
    return y.reshape(_B, _F, 24, 24), loss[0, 0]


# speedup vs baseline: 1.0025x; 1.0025x over previous
import functools

import jax
import jax.numpy as jnp
from jax import lax
from jax.experimental import pallas as pl
from jax.experimental.pallas import tpu as pltpu
from jax.experimental.pallas import tpu_sc as plsc

_B, _F, _NC, _P = 8, 256, 1024, 576
_SC_NC, _SC_NS, _L = 2, 16, 16
_NW = _SC_NC * _SC_NS
_FPW = _F // _NW
_PC = _P // _L


def _tc_body(x_ref, c_ref, y_ref, loss_ref, csq_ref):
    b = pl.program_id(0)
    nb = pl.num_programs(0)
    xb = x_ref[0]
    C = c_ref[...]

    @pl.when(b == 0)
    def _():
        csq_ref[...] = jnp.sum(C * C, axis=0, keepdims=True)

    c_sq = csq_ref[...]
    S2 = jax.lax.dot_general(C, xb, (((0,), (0,)), ((), ())),
                             preferred_element_type=jnp.float32)
    score = c_sq.T - 2.0 * S2
    m = jnp.min(score, axis=0, keepdims=True)
    iota0 = jax.lax.broadcasted_iota(jnp.int32, (_NC, _P), 0)
    idx = jnp.min(jnp.where(score == m, iota0, _NC), axis=0)
    oh = (iota0 == idx[None, :]).astype(jnp.float32)
    xq = jax.lax.dot_general(C, oh, (((1,), (0,)), ((), ())),
                             preferred_element_type=jnp.float32)
    y_ref[0] = xq
    x_sq = jnp.sum(xb * xb, axis=0)
    partial = jnp.sum(x_sq + m[0])

    @pl.when(b == 0)
    def _():
        loss_ref[0, 0] = 0.0

    loss_ref[0, 0] += partial

    @pl.when(b == nb - 1)
    def _():
        loss_ref[0, 0] = loss_ref[0, 0] / (_B * _F * _P)


def _sc_gather(c_hbm, idx_hbm, y_hbm, cflat, idxv, obuf):
    wid = lax.axis_index("s") * _SC_NC + lax.axis_index("c")
    f0 = wid * _FPW
    pltpu.sync_copy(c_hbm.at[pl.ds(f0 * _NC, _FPW * _NC)], cflat)
    pltpu.sync_copy(idx_hbm, idxv)

    def chunk(t, _):
        b = t // _PC
        c = t % _PC
        iv = idxv[b, pl.ds(c * _L, _L)]
        for fl in range(_FPW):
            vals = plsc.load_gather(cflat, [iv + fl * _NC])
            obuf[b, fl, pl.ds(c * _L, _L)] = vals
        return 0

    lax.fori_loop(0, _B * _PC, chunk, 0)
    pltpu.sync_copy(obuf, y_hbm.at[:, pl.ds(f0, _FPW), :])


def kernel(x, centroids):
    x3 = x.reshape(_B, _F, _P)
    y, loss = pl.pallas_call(
        _tc_body,
        grid=(_B,),
        in_specs=[
            pl.BlockSpec((1, _F, _P), lambda b: (b, 0, 0)),
            pl.BlockSpec((_F, _NC), lambda b: (0, 0)),
        ],
        out_specs=[
            pl.BlockSpec((1, _F, _P), lambda b: (b, 0, 0)),
            pl.BlockSpec(memory_space=pltpu.SMEM, block_shape=(1, 1),
                         index_map=lambda b: (0, 0)),
        ],
        out_shape=[
            jax.ShapeDtypeStruct((_B, _F, _P), jnp.float32),
            jax.ShapeDtypeStruct((1, 1), jnp.float32),
        ],
        scratch_shapes=[pltpu.VMEM((1, _NC), jnp.float32)],
        compiler_params=pltpu.CompilerParams(
            dimension_semantics=("arbitrary",),
        ),
    )(x3, centroids)
    return y.reshape(_B, _F, 24, 24), loss[0, 0]


# speedup vs baseline: 1.1218x; 1.1190x over previous
import jax
import jax.numpy as jnp
from jax.experimental import pallas as pl
from jax.experimental.pallas import tpu as pltpu

_B, _F, _NC, _P = 8, 256, 1024, 576
_NBS = 2
_NSTEP = _B // _NBS


def _tc_body(x_ref, c_ref, y_ref, loss_ref):
    g = pl.program_id(0)
    C = c_ref[...]
    c_sq = jnp.sum(C * C, axis=0, keepdims=True)
    iota0 = jax.lax.broadcasted_iota(jnp.int32, (_NC, _P), 0)
    partial = jnp.float32(0.0)
    for i in range(_NBS):
        xb = x_ref[i]
        S2 = jax.lax.dot_general(C, xb, (((0,), (0,)), ((), ())),
                                 preferred_element_type=jnp.float32)
        score = c_sq.T - 2.0 * S2
        m = jnp.min(score, axis=0, keepdims=True)
        idx = jnp.min(jnp.where(score == m, iota0, _NC), axis=0)
        oh = (iota0 == idx[None, :]).astype(jnp.float32)
        xq = jax.lax.dot_general(C, oh, (((1,), (0,)), ((), ())),
                                 preferred_element_type=jnp.float32)
        y_ref[i] = xq
        x_sq = jnp.sum(xb * xb, axis=0)
        partial += jnp.sum(x_sq + m[0])

    @pl.when(g == 0)
    def _():
        loss_ref[0, 0] = 0.0

    loss_ref[0, 0] += partial

    @pl.when(g == _NSTEP - 1)
    def _():
        loss_ref[0, 0] = loss_ref[0, 0] / (_B * _F * _P)


def kernel(x, centroids):
    x3 = x.reshape(_B, _F, _P)
    y, loss = pl.pallas_call(
        _tc_body,
        grid=(_NSTEP,),
        in_specs=[
            pl.BlockSpec((_NBS, _F, _P), lambda g: (g, 0, 0)),
            pl.BlockSpec((_F, _NC), lambda g: (0, 0)),
        ],
        out_specs=[
            pl.BlockSpec((_NBS, _F, _P), lambda g: (g, 0, 0)),
            pl.BlockSpec(memory_space=pltpu.SMEM, block_shape=(1, 1),
                         index_map=lambda g: (0, 0)),
        ],
        out_shape=[
            jax.ShapeDtypeStruct((_B, _F, _P), jnp.float32),
            jax.ShapeDtypeStruct((1, 1), jnp.float32),
        ],
        compiler_params=pltpu.CompilerParams(
            dimension_semantics=("arbitrary",),
        ),
    )(x3, centroids)
    return y.reshape(_B, _F, 24, 24), loss[0, 0]


# speedup vs baseline: 1.2430x; 1.1080x over previous
import jax
import jax.numpy as jnp
from jax.experimental import pallas as pl
from jax.experimental.pallas import tpu as pltpu

_B, _F, _NC, _P = 8, 256, 1024, 576
_NBS = 4
_NSTEP = _B // _NBS


def _tc_body(x_ref, c_ref, y_ref, loss_ref):
    g = pl.program_id(0)
    C = c_ref[...]
    c_sq = jnp.sum(C * C, axis=0, keepdims=True)
    iota0 = jax.lax.broadcasted_iota(jnp.int32, (_NC, _P), 0)
    partial = jnp.float32(0.0)
    for i in range(_NBS):
        xb = x_ref[i]
        S2 = jax.lax.dot_general(C, xb, (((0,), (0,)), ((), ())),
                                 preferred_element_type=jnp.float32)
        score = c_sq.T - 2.0 * S2
        idx = jnp.argmin(score, axis=0)
        oh = (iota0 == idx[None, :]).astype(jnp.float32)
        xq = jax.lax.dot_general(C, oh, (((1,), (0,)), ((), ())),
                                 preferred_element_type=jnp.float32)
        y_ref[i] = xq
        r = xb - xq
        partial += jnp.sum(r * r)

    @pl.when(g == 0)
    def _():
        loss_ref[0, 0] = 0.0

    loss_ref[0, 0] += partial

    @pl.when(g == _NSTEP - 1)
    def _():
        loss_ref[0, 0] = loss_ref[0, 0] / (_B * _F * _P)


def kernel(x, centroids):
    x3 = x.reshape(_B, _F, _P)
    y, loss = pl.pallas_call(
        _tc_body,
        grid=(_NSTEP,),
        in_specs=[
            pl.BlockSpec((_NBS, _F, _P), lambda g: (g, 0, 0)),
            pl.BlockSpec((_F, _NC), lambda g: (0, 0)),
        ],
        out_specs=[
            pl.BlockSpec((_NBS, _F, _P), lambda g: (g, 0, 0)),
            pl.BlockSpec(memory_space=pltpu.SMEM, block_shape=(1, 1),
                         index_map=lambda g: (0, 0)),
        ],
        out_shape=[
            jax.ShapeDtypeStruct((_B, _F, _P), jnp.float32),
            jax.ShapeDtypeStruct((1, 1), jnp.float32),
        ],
        compiler_params=pltpu.CompilerParams(
            dimension_semantics=("arbitrary",),
        ),
    )(x3, centroids)
    return y.reshape(_B, _F, 24, 24), loss[0, 0]
